# baseline (speedup 1.0000x reference)
import functools

import jax
import jax.numpy as jnp
from jax import lax
from jax.experimental import pallas as pl
from jax.experimental.pallas import tpu as pltpu
from jax.experimental.pallas import tpu_sc as plsc

NROWS = 64
NCOLS = 8192
KACT = 128
L = 16
SLICES = NCOLS // L

_MESH = plsc.VectorSubcoreMesh(core_axis_name="c", subcore_axis_name="s")

_I32MIN = -2147483648


def _mkkey(v):
    b = plsc.bitcast(v, jnp.int32)
    return b ^ ((b >> 31) | jnp.int32(_I32MIN))


def _bucket_hi(key):
    return plsc.bitcast(plsc.bitcast(key, jnp.uint32) >> jnp.uint32(24),
                        jnp.int32)


def _scan_hist2(hist_ref, rank0, rank1):
    lane = lax.iota(jnp.int32, L)
    comb0 = jnp.full((L,), -1, jnp.int32)
    comb1 = jnp.full((L,), -1, jnp.int32)
    acc0 = jnp.int32(0)
    acc1 = jnp.int32(0)
    for j in range(15, -1, -1):
        h0 = hist_ref[pl.ds(j * L, L)]
        h1 = hist_ref[pl.ds(256 + j * L, L)]
        r0 = lax.rev(h0, (0,))
        r1 = lax.rev(h1, (0,))
        c0 = plsc.cumsum(r0) + acc0
        c1 = plsc.cumsum(r1) + acc1
        bid = (j * L + L - 1) - lane
        comb0 = jnp.maximum(comb0, jnp.where(c0 >= rank0,
                                             (bid << 16) | (c0 - r0), -1))
        comb1 = jnp.maximum(comb1, jnp.where(c1 >= rank1,
                                             (bid << 16) | (c1 - r1), -1))
        acc0 = acc0 + jnp.sum(h0)
        acc1 = acc1 + jnp.sum(h1)
    best0 = jnp.max(comb0)
    best1 = jnp.max(comb1)
    return (best0 >> 16, rank0 - (best0 & 0xFFFF),
            best1 >> 16, rank1 - (best1 & 0xFFFF))


def _zero_hist(hist_ref):
    z = jnp.zeros((L,), jnp.int32)
    for j in range(32):
        hist_ref[pl.ds(j * L, L)] = z


@functools.partial(
    pl.kernel,
    out_type=jax.ShapeDtypeStruct((NROWS * NCOLS,), jnp.float32),
    mesh=_MESH,
    scratch_types=[
        pltpu.VMEM((2 * NCOLS,), jnp.float32),
        pltpu.VMEM((2 * NCOLS,), jnp.int32),
        pltpu.VMEM((2 * NCOLS,), jnp.int32),
        pltpu.VMEM((512,), jnp.int32),
        pltpu.SemaphoreType.DMA,
    ],
    compiler_params=pltpu.CompilerParams(needs_layout_passes=False),
)
def _sparsify_sc(x_hbm, out_hbm, xrow, canda, candb, hist, sem):
    wid = lax.axis_index("s") * 2 + lax.axis_index("c")
    ones = jnp.ones((L,), jnp.int32)
    lane = lax.iota(jnp.int32, L)

    base_hbm = wid * (2 * NCOLS)
    in_dma = pltpu.async_copy(x_hbm.at[pl.ds(base_hbm, 2 * NCOLS)], xrow,
                              sem)
    _zero_hist(hist)
    in_dma.wait()

    @plsc.parallel_loop(0, NCOLS, L, unroll=8)
    def h1(i):
        i0 = _bucket_hi(_mkkey(xrow[pl.ds(i, L)]))
        i1 = _bucket_hi(_mkkey(xrow[pl.ds(NCOLS + i, L)])) + 256
        plsc.addupdate_scatter(hist, [i0], ones)
        plsc.addupdate_scatter(hist, [i1], ones)

    bstar0, rank0, bstar1, rank1 = _scan_hist2(hist, jnp.int32(KACT),
                                               jnp.int32(KACT))
    z16 = jnp.zeros((L,), jnp.int32)
    m16 = jnp.full((L,), -1, jnp.int32)

    @plsc.parallel_loop(0, NCOLS, L, unroll=4, carry=(m16, m16))
    def c1(i, offs):
        off0, off1 = offs
        k0 = _mkkey(xrow[pl.ds(i, L)])
        k1 = _mkkey(xrow[pl.ds(NCOLS + i, L)])
        m0 = _bucket_hi(k0) == bstar0
        m1 = _bucket_hi(k1) == bstar1
        p0 = off0 + plsc.cumsum(ones, mask=m0)
        p1 = off1 + plsc.cumsum(ones, mask=m1)
        plsc.store_scatter(canda, [p0], k0, mask=m0)
        plsc.store_scatter(canda, [p1 + NCOLS], k1, mask=m1)
        return (off0 + plsc.all_reduce_population_count(m0),
                off1 + plsc.all_reduce_population_count(m1))

    nv0, nv1 = c1
    nv0 = nv0 + 1
    nv1 = nv1 + 1
    prefix0 = bstar0.astype(jnp.uint32) << jnp.uint32(24)
    prefix1 = bstar1.astype(jnp.uint32) << jnp.uint32(24)

    def radix_round(shift, src, dst, nv0, nv1, rank0, rank1, compact):
        sh = jnp.uint32(shift)
        hi = ((jnp.maximum(jnp.max(nv0), jnp.max(nv1)) + (L - 1)) // L) * L
        _zero_hist(hist)

        def ext(k):
            return plsc.bitcast((plsc.bitcast(k, jnp.uint32) >> sh)
                                & jnp.uint32(0xFF), jnp.int32)

        @plsc.parallel_loop(0, hi, L)
        def hr(i):
            i0 = ext(src[pl.ds(i, L)])
            i1 = ext(src[pl.ds(NCOLS + i, L)]) + 256
            iv = i + lane
            plsc.addupdate_scatter(hist, [i0], ones, mask=iv < nv0)
            plsc.addupdate_scatter(hist, [i1], ones, mask=iv < nv1)

        bstar0, rank0, bstar1, rank1 = _scan_hist2(hist, rank0, rank1)
        if not compact:
            return bstar0, rank0, bstar1, rank1, nv0, nv1

        @plsc.parallel_loop(0, hi, L, carry=(m16, m16))
        def cr(i, offs):
            off0, off1 = offs
            k0 = src[pl.ds(i, L)]
            k1 = src[pl.ds(NCOLS + i, L)]
            iv = i + lane
            m0 = jnp.logical_and(ext(k0) == bstar0, iv < nv0)
            m1 = jnp.logical_and(ext(k1) == bstar1, iv < nv1)
            p0 = off0 + plsc.cumsum(ones, mask=m0)
            p1 = off1 + plsc.cumsum(ones, mask=m1)
            plsc.store_scatter(dst, [p0], k0, mask=m0)
            plsc.store_scatter(dst, [p1 + NCOLS], k1, mask=m1)
            return (off0 + plsc.all_reduce_population_count(m0),
                    off1 + plsc.all_reduce_population_count(m1))

        nv0, nv1 = cr
        return bstar0, rank0, bstar1, rank1, nv0 + 1, nv1 + 1

    bstar0, rank0, bstar1, rank1, nv0, nv1 = radix_round(
        16, canda, candb, nv0, nv1, rank0, rank1, True)
    prefix0 = prefix0 | (bstar0.astype(jnp.uint32) << jnp.uint32(16))
    prefix1 = prefix1 | (bstar1.astype(jnp.uint32) << jnp.uint32(16))

    few = jnp.maximum(jnp.max(nv0), jnp.max(nv1)) <= L

    def sort_path():
        k0 = plsc.bitcast(candb[pl.ds(0, L)], jnp.uint32)
        k1 = plsc.bitcast(candb[pl.ds(NCOLS, L)], jnp.uint32)
        k0 = jnp.where(lane < nv0, k0, jnp.uint32(0))
        k1 = jnp.where(lane < nv1, k1, jnp.uint32(0))
        s0, _ = plsc.sort_key_val(k0, k0)
        s1, _ = plsc.sort_key_val(k1, k1)
        t0 = jnp.max(jnp.where(lane == L - rank0, s0, jnp.uint32(0)))
        t1 = jnp.max(jnp.where(lane == L - rank1, s1, jnp.uint32(0)))
        return t0, t1

    def radix_path():
        b0, r0, b1, r1, m0, m1 = radix_round(
            8, candb, canda, nv0, nv1, rank0, rank1, True)
        p0 = prefix0 | (b0.astype(jnp.uint32) << jnp.uint32(8))
        p1 = prefix1 | (b1.astype(jnp.uint32) << jnp.uint32(8))
        b0, r0, b1, r1, m0, m1 = radix_round(
            0, canda, candb, m0, m1, r0, r1, False)
        return p0 | b0.astype(jnp.uint32), p1 | b1.astype(jnp.uint32)

    thrkey0, thrkey1 = lax.cond(few, sort_path, radix_path)

    def unkey(key):
        bits = jnp.where(key >= jnp.uint32(0x80000000),
                         key ^ jnp.uint32(0x80000000),
                         key ^ jnp.uint32(0xFFFFFFFF))
        return lax.bitcast_convert_type(bits, jnp.float32)

    thr0 = unkey(thrkey0)
    thr1 = unkey(thrkey1)

    @plsc.parallel_loop(0, NCOLS, L, unroll=8)
    def fbody0(i):
        v0 = xrow[pl.ds(i, L)]
        xrow[pl.ds(i, L)] = jnp.where(v0 >= thr0, v0, jnp.float32(0.0))

    out_dma0 = pltpu.async_copy(
        xrow.at[pl.ds(0, NCOLS)], out_hbm.at[pl.ds(base_hbm, NCOLS)], sem)

    @plsc.parallel_loop(0, NCOLS, L, unroll=8)
    def fbody1(i):
        v1 = xrow[pl.ds(NCOLS + i, L)]
        xrow[pl.ds(NCOLS + i, L)] = jnp.where(v1 >= thr1, v1,
                                              jnp.float32(0.0))

    out_dma1 = pltpu.async_copy(
        xrow.at[pl.ds(NCOLS, NCOLS)],
        out_hbm.at[pl.ds(base_hbm + NCOLS, NCOLS)], sem)
    out_dma0.wait()
    out_dma1.wait()


@jax.jit
def kernel(x):
    out = _sparsify_sc(x.reshape(-1))
    return out.reshape(NROWS, NCOLS)


# speedup vs baseline: 1.0030x; 1.0030x over previous
import functools

import jax
import jax.numpy as jnp
from jax import lax
from jax.experimental import pallas as pl
from jax.experimental.pallas import tpu as pltpu
from jax.experimental.pallas import tpu_sc as plsc

NROWS = 64
NCOLS = 8192
KACT = 128
L = 16
SLICES = NCOLS // L

_MESH = plsc.VectorSubcoreMesh(core_axis_name="c", subcore_axis_name="s")

_I32MIN = -2147483648


def _mkkey(v):
    b = plsc.bitcast(v, jnp.int32)
    return b ^ ((b >> 31) | jnp.int32(_I32MIN))


def _bucket_hi(key):
    return plsc.bitcast(plsc.bitcast(key, jnp.uint32) >> jnp.uint32(24),
                        jnp.int32)


def _scan_hist2(hist_ref, rank0, rank1):
    lane = lax.iota(jnp.int32, L)
    comb0 = jnp.full((L,), -1, jnp.int32)
    comb1 = jnp.full((L,), -1, jnp.int32)
    acc0 = jnp.int32(0)
    acc1 = jnp.int32(0)
    for j in range(15, -1, -1):
        h0 = hist_ref[pl.ds(j * L, L)]
        h1 = hist_ref[pl.ds(256 + j * L, L)]
        r0 = lax.rev(h0, (0,))
        r1 = lax.rev(h1, (0,))
        c0 = plsc.cumsum(r0) + acc0
        c1 = plsc.cumsum(r1) + acc1
        bid = (j * L + L - 1) - lane
        comb0 = jnp.maximum(comb0, jnp.where(c0 >= rank0,
                                             (bid << 16) | (c0 - r0), -1))
        comb1 = jnp.maximum(comb1, jnp.where(c1 >= rank1,
                                             (bid << 16) | (c1 - r1), -1))
        acc0 = acc0 + jnp.sum(h0)
        acc1 = acc1 + jnp.sum(h1)
    best0 = jnp.max(comb0)
    best1 = jnp.max(comb1)
    return (best0 >> 16, rank0 - (best0 & 0xFFFF),
            best1 >> 16, rank1 - (best1 & 0xFFFF))


def _zero_hist(hist_ref):
    z = jnp.zeros((L,), jnp.int32)
    for j in range(32):
        hist_ref[pl.ds(j * L, L)] = z


@functools.partial(
    pl.kernel,
    out_type=jax.ShapeDtypeStruct((NROWS * NCOLS,), jnp.float32),
    mesh=_MESH,
    scratch_types=[
        pltpu.VMEM((2 * NCOLS,), jnp.float32),
        pltpu.VMEM((2 * NCOLS,), jnp.int32),
        pltpu.VMEM((2 * NCOLS,), jnp.int32),
        pltpu.VMEM((512,), jnp.int32),
        pltpu.SemaphoreType.DMA,
    ],
    compiler_params=pltpu.CompilerParams(needs_layout_passes=False),
)
def _sparsify_sc(x_hbm, out_hbm, xrow, canda, candb, hist, sem):
    wid = lax.axis_index("s") * 2 + lax.axis_index("c")
    ones = jnp.ones((L,), jnp.int32)
    lane = lax.iota(jnp.int32, L)

    base_hbm = wid * (2 * NCOLS)
    H = NCOLS // 2
    dma_a0 = pltpu.async_copy(x_hbm.at[pl.ds(base_hbm, H)],
                              xrow.at[pl.ds(0, H)], sem)
    dma_a1 = pltpu.async_copy(x_hbm.at[pl.ds(base_hbm + NCOLS, H)],
                              xrow.at[pl.ds(NCOLS, H)], sem)
    dma_b0 = pltpu.async_copy(x_hbm.at[pl.ds(base_hbm + H, H)],
                              xrow.at[pl.ds(H, H)], sem)
    dma_b1 = pltpu.async_copy(x_hbm.at[pl.ds(base_hbm + NCOLS + H, H)],
                              xrow.at[pl.ds(NCOLS + H, H)], sem)
    _zero_hist(hist)
    dma_a0.wait()
    dma_a1.wait()

    @plsc.parallel_loop(0, H, L, unroll=8)
    def h1a(i):
        i0 = _bucket_hi(_mkkey(xrow[pl.ds(i, L)]))
        i1 = _bucket_hi(_mkkey(xrow[pl.ds(NCOLS + i, L)])) + 256
        plsc.addupdate_scatter(hist, [i0], ones)
        plsc.addupdate_scatter(hist, [i1], ones)

    dma_b0.wait()
    dma_b1.wait()

    @plsc.parallel_loop(H, NCOLS, L, unroll=8)
    def h1b(i):
        i0 = _bucket_hi(_mkkey(xrow[pl.ds(i, L)]))
        i1 = _bucket_hi(_mkkey(xrow[pl.ds(NCOLS + i, L)])) + 256
        plsc.addupdate_scatter(hist, [i0], ones)
        plsc.addupdate_scatter(hist, [i1], ones)

    bstar0, rank0, bstar1, rank1 = _scan_hist2(hist, jnp.int32(KACT),
                                               jnp.int32(KACT))
    z16 = jnp.zeros((L,), jnp.int32)
    m16 = jnp.full((L,), -1, jnp.int32)

    @plsc.parallel_loop(0, NCOLS, L, unroll=8, carry=(m16, m16))
    def c1(i, offs):
        off0, off1 = offs
        k0 = _mkkey(xrow[pl.ds(i, L)])
        k1 = _mkkey(xrow[pl.ds(NCOLS + i, L)])
        m0 = _bucket_hi(k0) == bstar0
        m1 = _bucket_hi(k1) == bstar1
        p0 = off0 + plsc.cumsum(ones, mask=m0)
        p1 = off1 + plsc.cumsum(ones, mask=m1)
        plsc.store_scatter(canda, [p0], k0, mask=m0)
        plsc.store_scatter(canda, [p1 + NCOLS], k1, mask=m1)
        return (off0 + plsc.all_reduce_population_count(m0),
                off1 + plsc.all_reduce_population_count(m1))

    nv0, nv1 = c1
    nv0 = nv0 + 1
    nv1 = nv1 + 1
    prefix0 = bstar0.astype(jnp.uint32) << jnp.uint32(24)
    prefix1 = bstar1.astype(jnp.uint32) << jnp.uint32(24)

    def radix_round(shift, src, dst, nv0, nv1, rank0, rank1, compact):
        sh = jnp.uint32(shift)
        hi = ((jnp.maximum(jnp.max(nv0), jnp.max(nv1)) + (L - 1)) // L) * L
        _zero_hist(hist)

        def ext(k):
            return plsc.bitcast((plsc.bitcast(k, jnp.uint32) >> sh)
                                & jnp.uint32(0xFF), jnp.int32)

        @plsc.parallel_loop(0, hi, L)
        def hr(i):
            i0 = ext(src[pl.ds(i, L)])
            i1 = ext(src[pl.ds(NCOLS + i, L)]) + 256
            iv = i + lane
            plsc.addupdate_scatter(hist, [i0], ones, mask=iv < nv0)
            plsc.addupdate_scatter(hist, [i1], ones, mask=iv < nv1)

        bstar0, rank0, bstar1, rank1 = _scan_hist2(hist, rank0, rank1)
        if not compact:
            return bstar0, rank0, bstar1, rank1, nv0, nv1

        @plsc.parallel_loop(0, hi, L, carry=(m16, m16))
        def cr(i, offs):
            off0, off1 = offs
            k0 = src[pl.ds(i, L)]
            k1 = src[pl.ds(NCOLS + i, L)]
            iv = i + lane
            m0 = jnp.logical_and(ext(k0) == bstar0, iv < nv0)
            m1 = jnp.logical_and(ext(k1) == bstar1, iv < nv1)
            p0 = off0 + plsc.cumsum(ones, mask=m0)
            p1 = off1 + plsc.cumsum(ones, mask=m1)
            plsc.store_scatter(dst, [p0], k0, mask=m0)
            plsc.store_scatter(dst, [p1 + NCOLS], k1, mask=m1)
            return (off0 + plsc.all_reduce_population_count(m0),
                    off1 + plsc.all_reduce_population_count(m1))

        nv0, nv1 = cr
        return bstar0, rank0, bstar1, rank1, nv0 + 1, nv1 + 1

    bstar0, rank0, bstar1, rank1, nv0, nv1 = radix_round(
        16, canda, candb, nv0, nv1, rank0, rank1, True)
    prefix0 = prefix0 | (bstar0.astype(jnp.uint32) << jnp.uint32(16))
    prefix1 = prefix1 | (bstar1.astype(jnp.uint32) << jnp.uint32(16))

    few = jnp.maximum(jnp.max(nv0), jnp.max(nv1)) <= L

    def sort_path():
        k0 = plsc.bitcast(candb[pl.ds(0, L)], jnp.uint32)
        k1 = plsc.bitcast(candb[pl.ds(NCOLS, L)], jnp.uint32)
        k0 = jnp.where(lane < nv0, k0, jnp.uint32(0))
        k1 = jnp.where(lane < nv1, k1, jnp.uint32(0))
        s0, _ = plsc.sort_key_val(k0, k0)
        s1, _ = plsc.sort_key_val(k1, k1)
        t0 = jnp.max(jnp.where(lane == L - rank0, s0, jnp.uint32(0)))
        t1 = jnp.max(jnp.where(lane == L - rank1, s1, jnp.uint32(0)))
        return t0, t1

    def radix_path():
        b0, r0, b1, r1, m0, m1 = radix_round(
            8, candb, canda, nv0, nv1, rank0, rank1, True)
        p0 = prefix0 | (b0.astype(jnp.uint32) << jnp.uint32(8))
        p1 = prefix1 | (b1.astype(jnp.uint32) << jnp.uint32(8))
        b0, r0, b1, r1, m0, m1 = radix_round(
            0, canda, candb, m0, m1, r0, r1, False)
        return p0 | b0.astype(jnp.uint32), p1 | b1.astype(jnp.uint32)

    thrkey0, thrkey1 = lax.cond(few, sort_path, radix_path)

    def unkey(key):
        bits = jnp.where(key >= jnp.uint32(0x80000000),
                         key ^ jnp.uint32(0x80000000),
                         key ^ jnp.uint32(0xFFFFFFFF))
        return lax.bitcast_convert_type(bits, jnp.float32)

    thr0 = unkey(thrkey0)
    thr1 = unkey(thrkey1)

    @plsc.parallel_loop(0, NCOLS, L, unroll=8)
    def fbody0(i):
        v0 = xrow[pl.ds(i, L)]
        xrow[pl.ds(i, L)] = jnp.where(v0 >= thr0, v0, jnp.float32(0.0))

    out_dma0 = pltpu.async_copy(
        xrow.at[pl.ds(0, NCOLS)], out_hbm.at[pl.ds(base_hbm, NCOLS)], sem)

    @plsc.parallel_loop(0, NCOLS, L, unroll=8)
    def fbody1(i):
        v1 = xrow[pl.ds(NCOLS + i, L)]
        xrow[pl.ds(NCOLS + i, L)] = jnp.where(v1 >= thr1, v1,
                                              jnp.float32(0.0))

    out_dma1 = pltpu.async_copy(
        xrow.at[pl.ds(NCOLS, NCOLS)],
        out_hbm.at[pl.ds(base_hbm + NCOLS, NCOLS)], sem)
    out_dma0.wait()
    out_dma1.wait()


@jax.jit
def kernel(x):
    out = _sparsify_sc(x.reshape(-1))
    return out.reshape(NROWS, NCOLS)


# speedup vs baseline: 1.0047x; 1.0016x over previous
import functools

import jax
import jax.numpy as jnp
from jax import lax
from jax.experimental import pallas as pl
from jax.experimental.pallas import tpu as pltpu
from jax.experimental.pallas import tpu_sc as plsc

NROWS = 64
NCOLS = 8192
KACT = 128
L = 16
SLICES = NCOLS // L

_MESH = plsc.VectorSubcoreMesh(core_axis_name="c", subcore_axis_name="s")

_I32MIN = -2147483648


def _mkkey(v):
    b = plsc.bitcast(v, jnp.int32)
    return b ^ ((b >> 31) | jnp.int32(_I32MIN))


def _bucket_hi(key):
    return plsc.bitcast(plsc.bitcast(key, jnp.uint32) >> jnp.uint32(24),
                        jnp.int32)


def _scan_hist2(hist_ref, rank0, rank1):
    lane = lax.iota(jnp.int32, L)
    comb0 = jnp.full((L,), -1, jnp.int32)
    comb1 = jnp.full((L,), -1, jnp.int32)
    acc0 = jnp.int32(0)
    acc1 = jnp.int32(0)
    for j in range(15, -1, -1):
        h0 = hist_ref[pl.ds(j * L, L)]
        h1 = hist_ref[pl.ds(256 + j * L, L)]
        r0 = lax.rev(h0, (0,))
        r1 = lax.rev(h1, (0,))
        c0 = plsc.cumsum(r0) + acc0
        c1 = plsc.cumsum(r1) + acc1
        bid = (j * L + L - 1) - lane
        comb0 = jnp.maximum(comb0, jnp.where(c0 >= rank0,
                                             (bid << 16) | (c0 - r0), -1))
        comb1 = jnp.maximum(comb1, jnp.where(c1 >= rank1,
                                             (bid << 16) | (c1 - r1), -1))
        acc0 = acc0 + jnp.sum(h0)
        acc1 = acc1 + jnp.sum(h1)
    best0 = jnp.max(comb0)
    best1 = jnp.max(comb1)
    return (best0 >> 16, rank0 - (best0 & 0xFFFF),
            best1 >> 16, rank1 - (best1 & 0xFFFF))


def _zero_hist(hist_ref):
    z = jnp.zeros((L,), jnp.int32)
    for j in range(32):
        hist_ref[pl.ds(j * L, L)] = z


@functools.partial(
    pl.kernel,
    out_type=jax.ShapeDtypeStruct((NROWS * NCOLS,), jnp.float32),
    mesh=_MESH,
    scratch_types=[
        pltpu.VMEM((2 * NCOLS,), jnp.float32),
        pltpu.VMEM((2 * NCOLS,), jnp.int32),
        pltpu.VMEM((2 * NCOLS,), jnp.int32),
        pltpu.VMEM((512,), jnp.int32),
        pltpu.SemaphoreType.DMA,
    ],
    compiler_params=pltpu.CompilerParams(
        needs_layout_passes=False,
        disable_bounds_checks=True,
        disable_semaphore_checks=True,
        skip_device_barrier=True,
    ),
)
def _sparsify_sc(x_hbm, out_hbm, xrow, canda, candb, hist, sem):
    wid = lax.axis_index("s") * 2 + lax.axis_index("c")
    ones = jnp.ones((L,), jnp.int32)
    lane = lax.iota(jnp.int32, L)

    base_hbm = wid * (2 * NCOLS)
    H = NCOLS // 2
    dma_a0 = pltpu.async_copy(x_hbm.at[pl.ds(base_hbm, H)],
                              xrow.at[pl.ds(0, H)], sem)
    dma_a1 = pltpu.async_copy(x_hbm.at[pl.ds(base_hbm + NCOLS, H)],
                              xrow.at[pl.ds(NCOLS, H)], sem)
    dma_b0 = pltpu.async_copy(x_hbm.at[pl.ds(base_hbm + H, H)],
                              xrow.at[pl.ds(H, H)], sem)
    dma_b1 = pltpu.async_copy(x_hbm.at[pl.ds(base_hbm + NCOLS + H, H)],
                              xrow.at[pl.ds(NCOLS + H, H)], sem)
    _zero_hist(hist)
    dma_a0.wait()
    dma_a1.wait()

    @plsc.parallel_loop(0, H, L, unroll=8)
    def h1a(i):
        i0 = _bucket_hi(_mkkey(xrow[pl.ds(i, L)]))
        i1 = _bucket_hi(_mkkey(xrow[pl.ds(NCOLS + i, L)])) + 256
        plsc.addupdate_scatter(hist, [i0], ones)
        plsc.addupdate_scatter(hist, [i1], ones)

    dma_b0.wait()
    dma_b1.wait()

    @plsc.parallel_loop(H, NCOLS, L, unroll=8)
    def h1b(i):
        i0 = _bucket_hi(_mkkey(xrow[pl.ds(i, L)]))
        i1 = _bucket_hi(_mkkey(xrow[pl.ds(NCOLS + i, L)])) + 256
        plsc.addupdate_scatter(hist, [i0], ones)
        plsc.addupdate_scatter(hist, [i1], ones)

    bstar0, rank0, bstar1, rank1 = _scan_hist2(hist, jnp.int32(KACT),
                                               jnp.int32(KACT))
    z16 = jnp.zeros((L,), jnp.int32)
    m16 = jnp.full((L,), -1, jnp.int32)

    @plsc.parallel_loop(0, NCOLS, L, unroll=8, carry=(m16, m16))
    def c1(i, offs):
        off0, off1 = offs
        k0 = _mkkey(xrow[pl.ds(i, L)])
        k1 = _mkkey(xrow[pl.ds(NCOLS + i, L)])
        m0 = _bucket_hi(k0) == bstar0
        m1 = _bucket_hi(k1) == bstar1
        p0 = off0 + plsc.cumsum(ones, mask=m0)
        p1 = off1 + plsc.cumsum(ones, mask=m1)
        plsc.store_scatter(canda, [p0], k0, mask=m0)
        plsc.store_scatter(canda, [p1 + NCOLS], k1, mask=m1)
        return (off0 + plsc.all_reduce_population_count(m0),
                off1 + plsc.all_reduce_population_count(m1))

    nv0, nv1 = c1
    nv0 = nv0 + 1
    nv1 = nv1 + 1
    prefix0 = bstar0.astype(jnp.uint32) << jnp.uint32(24)
    prefix1 = bstar1.astype(jnp.uint32) << jnp.uint32(24)

    def radix_round(shift, src, dst, nv0, nv1, rank0, rank1, compact):
        sh = jnp.uint32(shift)
        hi = ((jnp.maximum(jnp.max(nv0), jnp.max(nv1)) + (L - 1)) // L) * L
        _zero_hist(hist)

        def ext(k):
            return plsc.bitcast((plsc.bitcast(k, jnp.uint32) >> sh)
                                & jnp.uint32(0xFF), jnp.int32)

        @plsc.parallel_loop(0, hi, L)
        def hr(i):
            i0 = ext(src[pl.ds(i, L)])
            i1 = ext(src[pl.ds(NCOLS + i, L)]) + 256
            iv = i + lane
            plsc.addupdate_scatter(hist, [i0], ones, mask=iv < nv0)
            plsc.addupdate_scatter(hist, [i1], ones, mask=iv < nv1)

        bstar0, rank0, bstar1, rank1 = _scan_hist2(hist, rank0, rank1)
        if not compact:
            return bstar0, rank0, bstar1, rank1, nv0, nv1

        @plsc.parallel_loop(0, hi, L, carry=(m16, m16))
        def cr(i, offs):
            off0, off1 = offs
            k0 = src[pl.ds(i, L)]
            k1 = src[pl.ds(NCOLS + i, L)]
            iv = i + lane
            m0 = jnp.logical_and(ext(k0) == bstar0, iv < nv0)
            m1 = jnp.logical_and(ext(k1) == bstar1, iv < nv1)
            p0 = off0 + plsc.cumsum(ones, mask=m0)
            p1 = off1 + plsc.cumsum(ones, mask=m1)
            plsc.store_scatter(dst, [p0], k0, mask=m0)
            plsc.store_scatter(dst, [p1 + NCOLS], k1, mask=m1)
            return (off0 + plsc.all_reduce_population_count(m0),
                    off1 + plsc.all_reduce_population_count(m1))

        nv0, nv1 = cr
        return bstar0, rank0, bstar1, rank1, nv0 + 1, nv1 + 1

    bstar0, rank0, bstar1, rank1, nv0, nv1 = radix_round(
        16, canda, candb, nv0, nv1, rank0, rank1, True)
    prefix0 = prefix0 | (bstar0.astype(jnp.uint32) << jnp.uint32(16))
    prefix1 = prefix1 | (bstar1.astype(jnp.uint32) << jnp.uint32(16))

    few = jnp.maximum(jnp.max(nv0), jnp.max(nv1)) <= L

    def sort_path():
        k0 = plsc.bitcast(candb[pl.ds(0, L)], jnp.uint32)
        k1 = plsc.bitcast(candb[pl.ds(NCOLS, L)], jnp.uint32)
        k0 = jnp.where(lane < nv0, k0, jnp.uint32(0))
        k1 = jnp.where(lane < nv1, k1, jnp.uint32(0))
        s0, _ = plsc.sort_key_val(k0, k0)
        s1, _ = plsc.sort_key_val(k1, k1)
        t0 = jnp.max(jnp.where(lane == L - rank0, s0, jnp.uint32(0)))
        t1 = jnp.max(jnp.where(lane == L - rank1, s1, jnp.uint32(0)))
        return t0, t1

    def radix_path():
        b0, r0, b1, r1, m0, m1 = radix_round(
            8, candb, canda, nv0, nv1, rank0, rank1, True)
        p0 = prefix0 | (b0.astype(jnp.uint32) << jnp.uint32(8))
        p1 = prefix1 | (b1.astype(jnp.uint32) << jnp.uint32(8))
        b0, r0, b1, r1, m0, m1 = radix_round(
            0, canda, candb, m0, m1, r0, r1, False)
        return p0 | b0.astype(jnp.uint32), p1 | b1.astype(jnp.uint32)

    thrkey0, thrkey1 = lax.cond(few, sort_path, radix_path)

    def unkey(key):
        bits = jnp.where(key >= jnp.uint32(0x80000000),
                         key ^ jnp.uint32(0x80000000),
                         key ^ jnp.uint32(0xFFFFFFFF))
        return lax.bitcast_convert_type(bits, jnp.float32)

    thr0 = unkey(thrkey0)
    thr1 = unkey(thrkey1)

    @plsc.parallel_loop(0, NCOLS, L, unroll=8)
    def fbody0(i):
        v0 = xrow[pl.ds(i, L)]
        xrow[pl.ds(i, L)] = jnp.where(v0 >= thr0, v0, jnp.float32(0.0))

    out_dma0 = pltpu.async_copy(
        xrow.at[pl.ds(0, NCOLS)], out_hbm.at[pl.ds(base_hbm, NCOLS)], sem)

    @plsc.parallel_loop(0, NCOLS, L, unroll=8)
    def fbody1(i):
        v1 = xrow[pl.ds(NCOLS + i, L)]
        xrow[pl.ds(NCOLS + i, L)] = jnp.where(v1 >= thr1, v1,
                                              jnp.float32(0.0))

    out_dma1 = pltpu.async_copy(
        xrow.at[pl.ds(NCOLS, NCOLS)],
        out_hbm.at[pl.ds(base_hbm + NCOLS, NCOLS)], sem)
    out_dma0.wait()
    out_dma1.wait()


@jax.jit
def kernel(x):
    out = _sparsify_sc(x.reshape(-1))
    return out.reshape(NROWS, NCOLS)
